# Initial kernel scaffold; baseline (speedup 1.0000x reference)
import jax
import jax.numpy as jnp
from jax.experimental import pallas as pl


def kernel(x, edge_index, edge_attr, batch, W_init, b_init, W1, b1, W2, b2, W3, b3, W_e2n, b_e2n):
    raise NotImplementedError("write your pallas kernel here")


# speedup vs baseline: 1.5480x; 1.5480x over previous
import functools

import jax
import jax.numpy as jnp
from jax import lax
from jax.experimental import pallas as pl
from jax.experimental.pallas import tpu as pltpu
from jax.experimental.pallas import tpu_sc as plsc

N_NODES = 10000
N_EDGES = 320000
D_NODE = 128
D_EDGE = 16
HIDDEN = 128
N_GRAPHS = 64

NC = 2
NS = 16
NW = NC * NS
EPW = N_EDGES // NW
K = 80
NCHUNK = EPW // K
RPT = 624
RTAIL = N_NODES - NS * RPT
ZR = 208
NZB = RPT // ZR

E2 = N_EDGES // 2
BEH = 2000
GRID_E = E2 // BEH
BN = 2000
GRID_N = N_NODES // BN


def _mesh():
    return plsc.VectorSubcoreMesh(core_axis_name="c", subcore_axis_name="s")


def _sc_gather_body(table, idx, out, idx_v, rows_v, sem):
    c = lax.axis_index("c")
    s = lax.axis_index("s")
    base = (s * NC + c) * EPW

    def chunk(j, carry):
        b = pl.multiple_of(base + j * K, K)
        pltpu.sync_copy(idx.at[pl.ds(b, K)], idx_v)
        pltpu.async_copy(table.at[idx_v], rows_v, sem).wait()
        pltpu.sync_copy(rows_v, out.at[pl.ds(b, K)])
        return carry

    lax.fori_loop(0, NCHUNK, chunk, 0)


def _sc_gather(table, idx):
    f = pl.kernel(
        _sc_gather_body,
        out_type=jax.ShapeDtypeStruct((N_EDGES, HIDDEN), jnp.float32),
        mesh=_mesh(),
        scratch_types=[
            pltpu.VMEM((K,), jnp.int32),
            pltpu.VMEM((K, HIDDEN), jnp.float32),
            pltpu.SemaphoreType.DMA,
        ],
    )
    return f(table, idx)


def _sc_scatter_body(vals, idx, zrows, out, idx_v, rows_v, zb_v, acc):
    c = lax.axis_index("c")
    s = lax.axis_index("s")
    base = (s * NC + c) * EPW
    rbase = s * RPT

    pltpu.sync_copy(zrows, zb_v)
    for j in range(NZB):
        pltpu.sync_copy(zb_v, acc.at[pl.ds(rbase + j * ZR, ZR)])

    @pl.when(s == NS - 1)
    def _zero_tail():
        pltpu.sync_copy(zb_v.at[pl.ds(0, RTAIL)], acc.at[pl.ds(NS * RPT, RTAIL)])

    plsc.subcore_barrier()

    def chunk(j, carry):
        b = pl.multiple_of(base + j * K, K)
        pltpu.sync_copy(idx.at[pl.ds(b, K)], idx_v)
        pltpu.sync_copy(vals.at[pl.ds(b, K)], rows_v)
        pltpu.sync_copy(rows_v, acc.at[idx_v], add=True)
        return carry

    lax.fori_loop(0, NCHUNK, chunk, 0)
    plsc.subcore_barrier()

    for j in range(NZB):
        pltpu.sync_copy(acc.at[pl.ds(rbase + j * ZR, ZR)], zb_v)
        pltpu.sync_copy(zb_v, out.at[c, pl.ds(rbase + j * ZR, ZR)])

    @pl.when(s == NS - 1)
    def _write_tail():
        pltpu.sync_copy(acc.at[pl.ds(NS * RPT, RTAIL)], zb_v.at[pl.ds(0, RTAIL)])
        pltpu.sync_copy(zb_v.at[pl.ds(0, RTAIL)], out.at[c, pl.ds(NS * RPT, RTAIL)])


def _sc_scatter(vals, idx, zrows):
    f = pl.kernel(
        _sc_scatter_body,
        out_type=jax.ShapeDtypeStruct((NC, N_NODES, HIDDEN), jnp.float32),
        mesh=_mesh(),
        scratch_types=[
            pltpu.VMEM((K,), jnp.int32),
            pltpu.VMEM((K, HIDDEN), jnp.float32),
            pltpu.VMEM((ZR, HIDDEN), jnp.float32),
            pltpu.VMEM_SHARED((N_NODES, HIDDEN), jnp.float32),
        ],
    )
    return f(vals, idx, zrows)


def _tc_init_body(xg_ref, ea_ref, Wx_ref, We_ref, b_ref, W1_ref, h0_ref, y_ref):
    h0 = jnp.maximum(
        jnp.dot(xg_ref[...], Wx_ref[...], preferred_element_type=jnp.float32)
        + jnp.dot(ea_ref[...], We_ref[...], preferred_element_type=jnp.float32)
        + b_ref[...],
        0.0,
    )
    h0_ref[...] = h0
    y_ref[...] = jnp.dot(h0, W1_ref[...], preferred_element_type=jnp.float32)


def _tc_init(xg2, ea2, Wxbd, Webd, bi2, W1bd):
    return pl.pallas_call(
        _tc_init_body,
        grid=(GRID_E,),
        in_specs=[
            pl.BlockSpec((BEH, 2 * D_NODE), lambda i: (i, 0)),
            pl.BlockSpec((BEH, 2 * D_EDGE), lambda i: (i, 0)),
            pl.BlockSpec((2 * D_NODE, 2 * HIDDEN), lambda i: (0, 0)),
            pl.BlockSpec((2 * D_EDGE, 2 * HIDDEN), lambda i: (0, 0)),
            pl.BlockSpec((1, 2 * HIDDEN), lambda i: (0, 0)),
            pl.BlockSpec((2 * HIDDEN, 2 * HIDDEN), lambda i: (0, 0)),
        ],
        out_specs=[
            pl.BlockSpec((BEH, 2 * HIDDEN), lambda i: (i, 0)),
            pl.BlockSpec((BEH, 2 * HIDDEN), lambda i: (i, 0)),
        ],
        out_shape=[
            jax.ShapeDtypeStruct((E2, 2 * HIDDEN), jnp.float32),
            jax.ShapeDtypeStruct((E2, 2 * HIDDEN), jnp.float32),
        ],
    )(xg2, ea2, Wxbd, Webd, bi2, W1bd)


def _tc_conv_body(g_ref, y_ref, h0_ref, b_ref, W_ref, out_ref):
    y = y_ref[...]
    swap = jnp.concatenate([y[:, HIDDEN:], y[:, :HIDDEN]], axis=1)
    h = jnp.maximum(g_ref[...] - swap + b_ref[...] + h0_ref[...], 0.0)
    out_ref[...] = jnp.dot(h, W_ref[...], preferred_element_type=jnp.float32)


def _tc_conv(g2, y2, h02, b2, Wbd):
    return pl.pallas_call(
        _tc_conv_body,
        grid=(GRID_E,),
        in_specs=[
            pl.BlockSpec((BEH, 2 * HIDDEN), lambda i: (i, 0)),
            pl.BlockSpec((BEH, 2 * HIDDEN), lambda i: (i, 0)),
            pl.BlockSpec((BEH, 2 * HIDDEN), lambda i: (i, 0)),
            pl.BlockSpec((1, 2 * HIDDEN), lambda i: (0, 0)),
            pl.BlockSpec((2 * HIDDEN, 2 * HIDDEN), lambda i: (0, 0)),
        ],
        out_specs=pl.BlockSpec((BEH, 2 * HIDDEN), lambda i: (i, 0)),
        out_shape=jax.ShapeDtypeStruct((E2, 2 * HIDDEN), jnp.float32),
    )(g2, y2, h02, b2, Wbd)


def _tc_convlast_body(g_ref, y_ref, h0_ref, b_ref, out_ref):
    y = y_ref[...]
    swap = jnp.concatenate([y[:, HIDDEN:], y[:, :HIDDEN]], axis=1)
    out_ref[...] = jnp.maximum(g_ref[...] - swap + b_ref[...] + h0_ref[...], 0.0)


def _tc_convlast(g2, y2, h02, b2):
    return pl.pallas_call(
        _tc_convlast_body,
        grid=(GRID_E,),
        in_specs=[
            pl.BlockSpec((BEH, 2 * HIDDEN), lambda i: (i, 0)),
            pl.BlockSpec((BEH, 2 * HIDDEN), lambda i: (i, 0)),
            pl.BlockSpec((BEH, 2 * HIDDEN), lambda i: (i, 0)),
            pl.BlockSpec((1, 2 * HIDDEN), lambda i: (0, 0)),
        ],
        out_specs=pl.BlockSpec((BEH, 2 * HIDDEN), lambda i: (i, 0)),
        out_shape=jax.ShapeDtypeStruct((E2, 2 * HIDDEN), jnp.float32),
    )(g2, y2, h02, b2)


def _tc_add_body(a_ref, b_ref, o_ref):
    o_ref[...] = a_ref[...] + b_ref[...]


def _tc_add(a, b):
    return pl.pallas_call(
        _tc_add_body,
        grid=(GRID_N,),
        in_specs=[
            pl.BlockSpec((BN, HIDDEN), lambda i: (i, 0)),
            pl.BlockSpec((BN, HIDDEN), lambda i: (i, 0)),
        ],
        out_specs=pl.BlockSpec((BN, HIDDEN), lambda i: (i, 0)),
        out_shape=jax.ShapeDtypeStruct((N_NODES, HIDDEN), jnp.float32),
    )(a, b)


def _tc_final_body(x_ref, p0_ref, p1_ref, bt_ref, At_ref, Bt_ref, be_ref, out_ref):
    s = p0_ref[...] + p1_ref[...]
    hn = jnp.maximum(
        jnp.dot(x_ref[...], At_ref[...], preferred_element_type=jnp.float32)
        + jnp.dot(s, Bt_ref[...], preferred_element_type=jnp.float32)
        + be_ref[...],
        0.0,
    )
    oh = (bt_ref[...] == lax.broadcasted_iota(jnp.int32, (BN, N_GRAPHS), 1)
          ).astype(jnp.float32)
    part = lax.dot_general(oh, hn, (((0,), (0,)), ((), ())),
                           preferred_element_type=jnp.float32)

    @pl.when(pl.program_id(0) == 0)
    def _():
        out_ref[...] = jnp.zeros_like(out_ref)

    out_ref[...] += part


def _tc_final(x, p0, p1, bt, At, Bt, be):
    return pl.pallas_call(
        _tc_final_body,
        grid=(GRID_N,),
        in_specs=[
            pl.BlockSpec((BN, D_NODE), lambda i: (i, 0)),
            pl.BlockSpec((BN, HIDDEN), lambda i: (i, 0)),
            pl.BlockSpec((BN, HIDDEN), lambda i: (i, 0)),
            pl.BlockSpec((BN, 1), lambda i: (i, 0)),
            pl.BlockSpec((D_NODE, HIDDEN), lambda i: (0, 0)),
            pl.BlockSpec((HIDDEN, HIDDEN), lambda i: (0, 0)),
            pl.BlockSpec((1, HIDDEN), lambda i: (0, 0)),
        ],
        out_specs=pl.BlockSpec((N_GRAPHS, HIDDEN), lambda i: (0, 0)),
        out_shape=jax.ShapeDtypeStruct((N_GRAPHS, HIDDEN), jnp.float32),
    )(x, p0, p1, bt, At, Bt, be)


def _blockdiag(Wt):
    i, o = Wt.shape
    z = jnp.zeros((2 * i, 2 * o), jnp.float32)
    return z.at[:i, :o].set(Wt).at[i:, o:].set(Wt)


def kernel(x, edge_index, edge_attr, batch, W_init, b_init, W1, b1, W2, b2,
           W3, b3, W_e2n, b_e2n):
    row = edge_index[0].astype(jnp.int32)
    col = edge_index[1].astype(jnp.int32)

    Wxbd = _blockdiag(W_init[:, :D_NODE].T)
    Webd = _blockdiag(W_init[:, D_NODE:].T)
    bi2 = jnp.tile(b_init, 2)[None, :]
    Wbds = (_blockdiag(W1.T), _blockdiag(W2.T), _blockdiag(W3.T))
    b2s = (jnp.tile(b1, 2)[None, :], jnp.tile(b2, 2)[None, :],
           jnp.tile(b3, 2)[None, :])
    zrows = jnp.zeros((ZR, HIDDEN), jnp.float32)

    xg = _sc_gather(x, row)
    h02, y = _tc_init(xg.reshape(E2, 2 * D_NODE),
                      edge_attr.reshape(E2, 2 * D_EDGE),
                      Wxbd, Webd, bi2, Wbds[0])

    h3_2 = None
    for i in range(3):
        part = _sc_scatter(y.reshape(N_EDGES, HIDDEN), col, zrows)
        agg = _tc_add(part[0], part[1])
        g = _sc_gather(agg, row)
        g2 = g.reshape(E2, 2 * HIDDEN)
        if i < 2:
            y = _tc_conv(g2, y, h02, b2s[i], Wbds[i + 1])
        else:
            h3_2 = _tc_convlast(g2, y, h02, b2s[i])

    part = _sc_scatter(h3_2.reshape(N_EDGES, HIDDEN), col, zrows)
    bt = batch.astype(jnp.int32).reshape(N_NODES, 1)
    emb = _tc_final(x, part[0], part[1], bt,
                    W_e2n[:, :D_NODE].T, W_e2n[:, D_NODE:].T, b_e2n[None, :])
    return emb


# speedup vs baseline: 2.1070x; 1.3611x over previous
import functools

import jax
import jax.numpy as jnp
from jax import lax
from jax.experimental import pallas as pl
from jax.experimental.pallas import tpu as pltpu
from jax.experimental.pallas import tpu_sc as plsc

N_NODES = 10000
N_EDGES = 320000
D_NODE = 128
D_EDGE = 16
HIDDEN = 128
N_GRAPHS = 64

NC = 2
NS = 16
NW = NC * NS
K = 128
IDXROWS = N_EDGES // K
IDXPAD = 2560
ROWS_PT = IDXPAD // NW
NPAIR = ROWS_PT // 2
RPT = 624
RTAIL = N_NODES - NS * RPT
ZR = 104
NZB = RPT // ZR

E2 = N_EDGES // 2
BEH = 2000
GRID_E = E2 // BEH
BN = 2000
GRID_N = N_NODES // BN


def _mesh():
    return plsc.VectorSubcoreMesh(core_axis_name="c", subcore_axis_name="s")


def _worker_bounds(c, s):
    w = s * NC + c
    irow = w * ROWS_PT
    npair = jnp.where(w == NW - 1, (IDXROWS - (NW - 1) * ROWS_PT) // 2, NPAIR)
    return w, irow, npair


def _sc_gather_body(table, idx2d, out, idxb, bufa, bufb, gsa, gsb, ssa, ssb):
    c = lax.axis_index("c")
    s = lax.axis_index("s")
    w, irow, npair = _worker_bounds(c, s)
    pltpu.sync_copy(idx2d.at[pl.ds(pl.multiple_of(irow, 8), ROWS_PT)], idxb)

    def ebase(j):
        return pl.multiple_of((irow + j) * K, K)

    def start_gather(j, buf, sem):
        pltpu.async_copy(table.at[idxb.at[j]], buf, sem)

    def start_store(j, buf, sem):
        pltpu.async_copy(buf, out.at[pl.ds(ebase(j), K)], sem)

    def drain_gather(buf, sem):
        pltpu.make_async_copy(table.at[pl.ds(0, K)], buf, sem).wait()

    def drain_store(buf, sem):
        pltpu.make_async_copy(buf, out.at[pl.ds(0, K)], sem).wait()

    start_gather(0, bufa, gsa)

    def pair(j2, carry):
        p = 2 * j2
        q = p + 1

        @pl.when(j2 > 0)
        def _():
            drain_store(bufb, ssb)

        start_gather(q, bufb, gsb)
        drain_gather(bufa, gsa)
        start_store(p, bufa, ssa)

        @pl.when(j2 < npair - 1)
        def _():
            drain_store(bufa, ssa)
            start_gather(p + 2, bufa, gsa)

        drain_gather(bufb, gsb)
        start_store(q, bufb, ssb)
        return carry

    lax.fori_loop(0, npair, pair, 0)
    drain_store(bufa, ssa)
    drain_store(bufb, ssb)


def _sc_gather(table, idx2d):
    f = pl.kernel(
        _sc_gather_body,
        out_type=jax.ShapeDtypeStruct((N_EDGES, HIDDEN), jnp.float32),
        mesh=_mesh(),
        scratch_types=[
            pltpu.VMEM((ROWS_PT, K), jnp.int32),
            pltpu.VMEM((K, HIDDEN), jnp.float32),
            pltpu.VMEM((K, HIDDEN), jnp.float32),
            pltpu.SemaphoreType.DMA,
            pltpu.SemaphoreType.DMA,
            pltpu.SemaphoreType.DMA,
            pltpu.SemaphoreType.DMA,
        ],
    )
    return f(table, idx2d)


def _sc_scatter_body(vals, idx2d, zrows, out, idxb, bufa, bufb, vsa, vsb, acc):
    c = lax.axis_index("c")
    s = lax.axis_index("s")
    w, irow, npair = _worker_bounds(c, s)
    rbase = s * RPT

    pltpu.sync_copy(idx2d.at[pl.ds(pl.multiple_of(irow, 8), ROWS_PT)], idxb)
    pltpu.sync_copy(zrows, bufa.at[pl.ds(0, ZR)])
    for j in range(NZB):
        pltpu.sync_copy(bufa.at[pl.ds(0, ZR)], acc.at[pl.ds(rbase + j * ZR, ZR)])

    @pl.when(s == NS - 1)
    def _zero_tail():
        pltpu.sync_copy(bufa.at[pl.ds(0, RTAIL)], acc.at[pl.ds(NS * RPT, RTAIL)])

    plsc.subcore_barrier()

    def ebase(j):
        return pl.multiple_of((irow + j) * K, K)

    def start_load(j, buf, sem):
        pltpu.async_copy(vals.at[pl.ds(ebase(j), K)], buf, sem)

    def drain_load(buf, sem):
        pltpu.make_async_copy(vals.at[pl.ds(0, K)], buf, sem).wait()

    start_load(0, bufa, vsa)

    def pair(j2, carry):
        p = 2 * j2
        q = p + 1
        start_load(q, bufb, vsb)
        drain_load(bufa, vsa)
        pltpu.sync_copy(bufa, acc.at[idxb.at[p]], add=True)

        @pl.when(j2 < npair - 1)
        def _():
            start_load(p + 2, bufa, vsa)

        drain_load(bufb, vsb)
        pltpu.sync_copy(bufb, acc.at[idxb.at[q]], add=True)
        return carry

    lax.fori_loop(0, npair, pair, 0)
    plsc.subcore_barrier()

    for j in range(NZB):
        pltpu.sync_copy(acc.at[pl.ds(rbase + j * ZR, ZR)], bufa.at[pl.ds(0, ZR)])
        pltpu.sync_copy(bufa.at[pl.ds(0, ZR)], out.at[c, pl.ds(rbase + j * ZR, ZR)])

    @pl.when(s == NS - 1)
    def _write_tail():
        pltpu.sync_copy(acc.at[pl.ds(NS * RPT, RTAIL)], bufa.at[pl.ds(0, RTAIL)])
        pltpu.sync_copy(bufa.at[pl.ds(0, RTAIL)], out.at[c, pl.ds(NS * RPT, RTAIL)])


def _sc_scatter(vals, idx2d, zrows):
    f = pl.kernel(
        _sc_scatter_body,
        out_type=jax.ShapeDtypeStruct((NC, N_NODES, HIDDEN), jnp.float32),
        mesh=_mesh(),
        scratch_types=[
            pltpu.VMEM((ROWS_PT, K), jnp.int32),
            pltpu.VMEM((K, HIDDEN), jnp.float32),
            pltpu.VMEM((K, HIDDEN), jnp.float32),
            pltpu.SemaphoreType.DMA,
            pltpu.SemaphoreType.DMA,
            pltpu.VMEM_SHARED((N_NODES, HIDDEN), jnp.float32),
        ],
    )
    return f(vals, idx2d, zrows)


def _tc_init_body(xg_ref, ea_ref, Wx_ref, We_ref, b_ref, W1_ref, h0_ref, y_ref):
    h0 = jnp.maximum(
        jnp.dot(xg_ref[...], Wx_ref[...], preferred_element_type=jnp.float32)
        + jnp.dot(ea_ref[...], We_ref[...], preferred_element_type=jnp.float32)
        + b_ref[...],
        0.0,
    )
    h0_ref[...] = h0
    y_ref[...] = jnp.dot(h0, W1_ref[...], preferred_element_type=jnp.float32)


def _tc_init(xg2, ea2, Wxbd, Webd, bi2, W1bd):
    return pl.pallas_call(
        _tc_init_body,
        grid=(GRID_E,),
        in_specs=[
            pl.BlockSpec((BEH, 2 * D_NODE), lambda i: (i, 0)),
            pl.BlockSpec((BEH, 2 * D_EDGE), lambda i: (i, 0)),
            pl.BlockSpec((2 * D_NODE, 2 * HIDDEN), lambda i: (0, 0)),
            pl.BlockSpec((2 * D_EDGE, 2 * HIDDEN), lambda i: (0, 0)),
            pl.BlockSpec((1, 2 * HIDDEN), lambda i: (0, 0)),
            pl.BlockSpec((2 * HIDDEN, 2 * HIDDEN), lambda i: (0, 0)),
        ],
        out_specs=[
            pl.BlockSpec((BEH, 2 * HIDDEN), lambda i: (i, 0)),
            pl.BlockSpec((BEH, 2 * HIDDEN), lambda i: (i, 0)),
        ],
        out_shape=[
            jax.ShapeDtypeStruct((E2, 2 * HIDDEN), jnp.float32),
            jax.ShapeDtypeStruct((E2, 2 * HIDDEN), jnp.float32),
        ],
    )(xg2, ea2, Wxbd, Webd, bi2, W1bd)


def _tc_conv_body(g_ref, y_ref, h0_ref, b_ref, W_ref, out_ref):
    y = y_ref[...]
    swap = jnp.concatenate([y[:, HIDDEN:], y[:, :HIDDEN]], axis=1)
    h = jnp.maximum(g_ref[...] - swap + b_ref[...] + h0_ref[...], 0.0)
    out_ref[...] = jnp.dot(h, W_ref[...], preferred_element_type=jnp.float32)


def _tc_conv(g2, y2, h02, b2, Wbd):
    return pl.pallas_call(
        _tc_conv_body,
        grid=(GRID_E,),
        in_specs=[
            pl.BlockSpec((BEH, 2 * HIDDEN), lambda i: (i, 0)),
            pl.BlockSpec((BEH, 2 * HIDDEN), lambda i: (i, 0)),
            pl.BlockSpec((BEH, 2 * HIDDEN), lambda i: (i, 0)),
            pl.BlockSpec((1, 2 * HIDDEN), lambda i: (0, 0)),
            pl.BlockSpec((2 * HIDDEN, 2 * HIDDEN), lambda i: (0, 0)),
        ],
        out_specs=pl.BlockSpec((BEH, 2 * HIDDEN), lambda i: (i, 0)),
        out_shape=jax.ShapeDtypeStruct((E2, 2 * HIDDEN), jnp.float32),
    )(g2, y2, h02, b2, Wbd)


def _tc_convlast_body(g_ref, y_ref, h0_ref, b_ref, out_ref):
    y = y_ref[...]
    swap = jnp.concatenate([y[:, HIDDEN:], y[:, :HIDDEN]], axis=1)
    out_ref[...] = jnp.maximum(g_ref[...] - swap + b_ref[...] + h0_ref[...], 0.0)


def _tc_convlast(g2, y2, h02, b2):
    return pl.pallas_call(
        _tc_convlast_body,
        grid=(GRID_E,),
        in_specs=[
            pl.BlockSpec((BEH, 2 * HIDDEN), lambda i: (i, 0)),
            pl.BlockSpec((BEH, 2 * HIDDEN), lambda i: (i, 0)),
            pl.BlockSpec((BEH, 2 * HIDDEN), lambda i: (i, 0)),
            pl.BlockSpec((1, 2 * HIDDEN), lambda i: (0, 0)),
        ],
        out_specs=pl.BlockSpec((BEH, 2 * HIDDEN), lambda i: (i, 0)),
        out_shape=jax.ShapeDtypeStruct((E2, 2 * HIDDEN), jnp.float32),
    )(g2, y2, h02, b2)


def _tc_add_body(a_ref, b_ref, o_ref):
    o_ref[...] = a_ref[...] + b_ref[...]


def _tc_add(a, b):
    return pl.pallas_call(
        _tc_add_body,
        grid=(GRID_N,),
        in_specs=[
            pl.BlockSpec((BN, HIDDEN), lambda i: (i, 0)),
            pl.BlockSpec((BN, HIDDEN), lambda i: (i, 0)),
        ],
        out_specs=pl.BlockSpec((BN, HIDDEN), lambda i: (i, 0)),
        out_shape=jax.ShapeDtypeStruct((N_NODES, HIDDEN), jnp.float32),
    )(a, b)


def _tc_final_body(x_ref, p0_ref, p1_ref, bt_ref, At_ref, Bt_ref, be_ref, out_ref):
    s = p0_ref[...] + p1_ref[...]
    hn = jnp.maximum(
        jnp.dot(x_ref[...], At_ref[...], preferred_element_type=jnp.float32)
        + jnp.dot(s, Bt_ref[...], preferred_element_type=jnp.float32)
        + be_ref[...],
        0.0,
    )
    oh = (bt_ref[...] == lax.broadcasted_iota(jnp.int32, (BN, N_GRAPHS), 1)
          ).astype(jnp.float32)
    part = lax.dot_general(oh, hn, (((0,), (0,)), ((), ())),
                           preferred_element_type=jnp.float32)

    @pl.when(pl.program_id(0) == 0)
    def _():
        out_ref[...] = jnp.zeros_like(out_ref)

    out_ref[...] += part


def _tc_final(x, p0, p1, bt, At, Bt, be):
    return pl.pallas_call(
        _tc_final_body,
        grid=(GRID_N,),
        in_specs=[
            pl.BlockSpec((BN, D_NODE), lambda i: (i, 0)),
            pl.BlockSpec((BN, HIDDEN), lambda i: (i, 0)),
            pl.BlockSpec((BN, HIDDEN), lambda i: (i, 0)),
            pl.BlockSpec((BN, 1), lambda i: (i, 0)),
            pl.BlockSpec((D_NODE, HIDDEN), lambda i: (0, 0)),
            pl.BlockSpec((HIDDEN, HIDDEN), lambda i: (0, 0)),
            pl.BlockSpec((1, HIDDEN), lambda i: (0, 0)),
        ],
        out_specs=pl.BlockSpec((N_GRAPHS, HIDDEN), lambda i: (0, 0)),
        out_shape=jax.ShapeDtypeStruct((N_GRAPHS, HIDDEN), jnp.float32),
    )(x, p0, p1, bt, At, Bt, be)


def _blockdiag(Wt):
    i, o = Wt.shape
    z = jnp.zeros((2 * i, 2 * o), jnp.float32)
    return z.at[:i, :o].set(Wt).at[i:, o:].set(Wt)


def kernel(x, edge_index, edge_attr, batch, W_init, b_init, W1, b1, W2, b2,
           W3, b3, W_e2n, b_e2n):
    row = edge_index[0].astype(jnp.int32)
    col = edge_index[1].astype(jnp.int32)
    pad = ((0, IDXPAD - IDXROWS), (0, 0))
    row2d = jnp.pad(row.reshape(IDXROWS, K), pad)
    col2d = jnp.pad(col.reshape(IDXROWS, K), pad)

    Wxbd = _blockdiag(W_init[:, :D_NODE].T)
    Webd = _blockdiag(W_init[:, D_NODE:].T)
    bi2 = jnp.tile(b_init, 2)[None, :]
    Wbds = (_blockdiag(W1.T), _blockdiag(W2.T), _blockdiag(W3.T))
    b2s = (jnp.tile(b1, 2)[None, :], jnp.tile(b2, 2)[None, :],
           jnp.tile(b3, 2)[None, :])
    zrows = jnp.zeros((ZR, HIDDEN), jnp.float32)

    xg = _sc_gather(x, row2d)
    h02, y = _tc_init(xg.reshape(E2, 2 * D_NODE),
                      edge_attr.reshape(E2, 2 * D_EDGE),
                      Wxbd, Webd, bi2, Wbds[0])

    h3_2 = None
    for i in range(3):
        part = _sc_scatter(y.reshape(N_EDGES, HIDDEN), col2d, zrows)
        agg = _tc_add(part[0], part[1])
        g = _sc_gather(agg, row2d)
        g2 = g.reshape(E2, 2 * HIDDEN)
        if i < 2:
            y = _tc_conv(g2, y, h02, b2s[i], Wbds[i + 1])
        else:
            h3_2 = _tc_convlast(g2, y, h02, b2s[i])

    part = _sc_scatter(h3_2.reshape(N_EDGES, HIDDEN), col2d, zrows)
    bt = batch.astype(jnp.int32).reshape(N_NODES, 1)
    emb = _tc_final(x, part[0], part[1], bt,
                    W_e2n[:, :D_NODE].T, W_e2n[:, D_NODE:].T, b_e2n[None, :])
    return emb


# speedup vs baseline: 3.3137x; 1.5727x over previous
import functools

import jax
import jax.numpy as jnp
from jax import lax
from jax.experimental import pallas as pl
from jax.experimental.pallas import tpu as pltpu
from jax.experimental.pallas import tpu_sc as plsc

N_NODES = 10000
N_EDGES = 320000
D_NODE = 128
D_EDGE = 16
HIDDEN = 128
N_GRAPHS = 64

NC = 2
NS = 16
NW = NC * NS
K = 128
IDXROWS = N_EDGES // K
IDXPAD = 2560
ROWS_PT = IDXPAD // NW
NPAIR = ROWS_PT // 2
RPT = 624
RTAIL = N_NODES - NS * RPT
ZR = 104
NZB = RPT // ZR

BE = 2000
GRID_E = N_EDGES // BE
BN = 2000
GRID_N = N_NODES // BN


def _mesh():
    return plsc.VectorSubcoreMesh(core_axis_name="c", subcore_axis_name="s")


def _worker_bounds(c, s):
    w = s * NC + c
    irow = w * ROWS_PT
    npair = jnp.where(w == NW - 1, (IDXROWS - (NW - 1) * ROWS_PT) // 2, NPAIR)
    return w, irow, npair


def _sc_gather_body(table, idx2d, out, idxb, bufa, bufb, gsa, gsb, ssa, ssb):
    c = lax.axis_index("c")
    s = lax.axis_index("s")
    w, irow, npair = _worker_bounds(c, s)
    pltpu.sync_copy(idx2d.at[pl.ds(pl.multiple_of(irow, 8), ROWS_PT)], idxb)

    def ebase(j):
        return pl.multiple_of((irow + j) * K, K)

    def start_gather(j, buf, sem):
        pltpu.async_copy(table.at[idxb.at[j]], buf, sem)

    def start_store(j, buf, sem):
        pltpu.async_copy(buf, out.at[pl.ds(ebase(j), K)], sem)

    def drain_gather(buf, sem):
        pltpu.make_async_copy(table.at[pl.ds(0, K)], buf, sem).wait()

    def drain_store(buf, sem):
        pltpu.make_async_copy(buf, out.at[pl.ds(0, K)], sem).wait()

    start_gather(0, bufa, gsa)

    def pair(j2, carry):
        p = 2 * j2
        q = p + 1

        @pl.when(j2 > 0)
        def _():
            drain_store(bufb, ssb)

        start_gather(q, bufb, gsb)
        drain_gather(bufa, gsa)
        start_store(p, bufa, ssa)

        @pl.when(j2 < npair - 1)
        def _():
            drain_store(bufa, ssa)
            start_gather(p + 2, bufa, gsa)

        drain_gather(bufb, gsb)
        start_store(q, bufb, ssb)
        return carry

    lax.fori_loop(0, npair, pair, 0)
    drain_store(bufa, ssa)
    drain_store(bufb, ssb)


def _sc_gather(table, idx2d):
    f = pl.kernel(
        _sc_gather_body,
        out_type=jax.ShapeDtypeStruct((N_EDGES, HIDDEN), jnp.float32),
        mesh=_mesh(),
        scratch_types=[
            pltpu.VMEM((ROWS_PT, K), jnp.int32),
            pltpu.VMEM((K, HIDDEN), jnp.float32),
            pltpu.VMEM((K, HIDDEN), jnp.float32),
            pltpu.SemaphoreType.DMA,
            pltpu.SemaphoreType.DMA,
            pltpu.SemaphoreType.DMA,
            pltpu.SemaphoreType.DMA,
        ],
    )
    return f(table, idx2d)


def _sc_scatter_body(vals, idx2d, zrows, out, idxb, bufa, bufb, vsa, vsb, acc):
    c = lax.axis_index("c")
    s = lax.axis_index("s")
    w, irow, npair = _worker_bounds(c, s)
    rbase = s * RPT

    pltpu.sync_copy(idx2d.at[pl.ds(pl.multiple_of(irow, 8), ROWS_PT)], idxb)
    pltpu.sync_copy(zrows, bufa.at[pl.ds(0, ZR)])
    for j in range(NZB):
        pltpu.sync_copy(bufa.at[pl.ds(0, ZR)], acc.at[pl.ds(rbase + j * ZR, ZR)])

    @pl.when(s == NS - 1)
    def _zero_tail():
        pltpu.sync_copy(bufa.at[pl.ds(0, RTAIL)], acc.at[pl.ds(NS * RPT, RTAIL)])

    plsc.subcore_barrier()

    def ebase(j):
        return pl.multiple_of((irow + j) * K, K)

    def start_load(j, buf, sem):
        pltpu.async_copy(vals.at[pl.ds(ebase(j), K)], buf, sem)

    def drain_load(buf, sem):
        pltpu.make_async_copy(vals.at[pl.ds(0, K)], buf, sem).wait()

    start_load(0, bufa, vsa)

    def pair(j2, carry):
        p = 2 * j2
        q = p + 1
        start_load(q, bufb, vsb)
        drain_load(bufa, vsa)
        pltpu.sync_copy(bufa, acc.at[idxb.at[p]], add=True)

        @pl.when(j2 < npair - 1)
        def _():
            start_load(p + 2, bufa, vsa)

        drain_load(bufb, vsb)
        pltpu.sync_copy(bufb, acc.at[idxb.at[q]], add=True)
        return carry

    lax.fori_loop(0, npair, pair, 0)
    plsc.subcore_barrier()

    for j in range(NZB):
        pltpu.sync_copy(acc.at[pl.ds(rbase + j * ZR, ZR)], bufa.at[pl.ds(0, ZR)])
        pltpu.sync_copy(bufa.at[pl.ds(0, ZR)], out.at[c, pl.ds(rbase + j * ZR, ZR)])

    @pl.when(s == NS - 1)
    def _write_tail():
        pltpu.sync_copy(acc.at[pl.ds(NS * RPT, RTAIL)], bufa.at[pl.ds(0, RTAIL)])
        pltpu.sync_copy(bufa.at[pl.ds(0, RTAIL)], out.at[c, pl.ds(NS * RPT, RTAIL)])


def _sc_scatter(vals, idx2d, zrows):
    f = pl.kernel(
        _sc_scatter_body,
        out_type=jax.ShapeDtypeStruct((NC, N_NODES, HIDDEN), jnp.float32),
        mesh=_mesh(),
        scratch_types=[
            pltpu.VMEM((ROWS_PT, K), jnp.int32),
            pltpu.VMEM((K, HIDDEN), jnp.float32),
            pltpu.VMEM((K, HIDDEN), jnp.float32),
            pltpu.SemaphoreType.DMA,
            pltpu.SemaphoreType.DMA,
            pltpu.VMEM_SHARED((N_NODES, HIDDEN), jnp.float32),
        ],
    )
    return f(vals, idx2d, zrows)


def _pair_swap(y):
    even = (lax.broadcasted_iota(jnp.int32, y.shape, 0) & 1) == 0
    return jnp.where(even, pltpu.roll(y, y.shape[0] - 1, 0),
                     pltpu.roll(y, 1, 0))


def _tc_init_body(xg_ref, ea_ref, Wx_ref, We_ref, b_ref, W1_ref, h0_ref, y_ref):
    h0 = jnp.maximum(
        jnp.dot(xg_ref[...], Wx_ref[...], preferred_element_type=jnp.float32)
        + jnp.dot(ea_ref[...], We_ref[...], preferred_element_type=jnp.float32)
        + b_ref[...],
        0.0,
    )
    h0_ref[...] = h0
    y_ref[...] = jnp.dot(h0, W1_ref[...], preferred_element_type=jnp.float32)


def _tc_init(xg, ea, Wxt, Wet, bi, W1t):
    return pl.pallas_call(
        _tc_init_body,
        grid=(GRID_E,),
        in_specs=[
            pl.BlockSpec((BE, D_NODE), lambda i: (i, 0)),
            pl.BlockSpec((BE, D_EDGE), lambda i: (i, 0)),
            pl.BlockSpec((D_NODE, HIDDEN), lambda i: (0, 0)),
            pl.BlockSpec((D_EDGE, HIDDEN), lambda i: (0, 0)),
            pl.BlockSpec((1, HIDDEN), lambda i: (0, 0)),
            pl.BlockSpec((HIDDEN, HIDDEN), lambda i: (0, 0)),
        ],
        out_specs=[
            pl.BlockSpec((BE, HIDDEN), lambda i: (i, 0)),
            pl.BlockSpec((BE, HIDDEN), lambda i: (i, 0)),
        ],
        out_shape=[
            jax.ShapeDtypeStruct((N_EDGES, HIDDEN), jnp.float32),
            jax.ShapeDtypeStruct((N_EDGES, HIDDEN), jnp.float32),
        ],
    )(xg, ea, Wxt, Wet, bi, W1t)


def _tc_conv_body(g_ref, y_ref, h0_ref, b_ref, W_ref, out_ref):
    h = jnp.maximum(
        g_ref[...] - _pair_swap(y_ref[...]) + b_ref[...] + h0_ref[...], 0.0)
    out_ref[...] = jnp.dot(h, W_ref[...], preferred_element_type=jnp.float32)


def _tc_conv(g, y, h0, b, Wt):
    return pl.pallas_call(
        _tc_conv_body,
        grid=(GRID_E,),
        in_specs=[
            pl.BlockSpec((BE, HIDDEN), lambda i: (i, 0)),
            pl.BlockSpec((BE, HIDDEN), lambda i: (i, 0)),
            pl.BlockSpec((BE, HIDDEN), lambda i: (i, 0)),
            pl.BlockSpec((1, HIDDEN), lambda i: (0, 0)),
            pl.BlockSpec((HIDDEN, HIDDEN), lambda i: (0, 0)),
        ],
        out_specs=pl.BlockSpec((BE, HIDDEN), lambda i: (i, 0)),
        out_shape=jax.ShapeDtypeStruct((N_EDGES, HIDDEN), jnp.float32),
    )(g, y, h0, b, Wt)


def _tc_convlast_body(g_ref, y_ref, h0_ref, b_ref, out_ref):
    out_ref[...] = jnp.maximum(
        g_ref[...] - _pair_swap(y_ref[...]) + b_ref[...] + h0_ref[...], 0.0)


def _tc_convlast(g, y, h0, b):
    return pl.pallas_call(
        _tc_convlast_body,
        grid=(GRID_E,),
        in_specs=[
            pl.BlockSpec((BE, HIDDEN), lambda i: (i, 0)),
            pl.BlockSpec((BE, HIDDEN), lambda i: (i, 0)),
            pl.BlockSpec((BE, HIDDEN), lambda i: (i, 0)),
            pl.BlockSpec((1, HIDDEN), lambda i: (0, 0)),
        ],
        out_specs=pl.BlockSpec((BE, HIDDEN), lambda i: (i, 0)),
        out_shape=jax.ShapeDtypeStruct((N_EDGES, HIDDEN), jnp.float32),
    )(g, y, h0, b)


def _tc_add_body(a_ref, b_ref, o_ref):
    o_ref[...] = a_ref[...] + b_ref[...]


def _tc_add(a, b):
    return pl.pallas_call(
        _tc_add_body,
        grid=(GRID_N,),
        in_specs=[
            pl.BlockSpec((BN, HIDDEN), lambda i: (i, 0)),
            pl.BlockSpec((BN, HIDDEN), lambda i: (i, 0)),
        ],
        out_specs=pl.BlockSpec((BN, HIDDEN), lambda i: (i, 0)),
        out_shape=jax.ShapeDtypeStruct((N_NODES, HIDDEN), jnp.float32),
    )(a, b)


def _tc_final_body(x_ref, p0_ref, p1_ref, bt_ref, At_ref, Bt_ref, be_ref, out_ref):
    s = p0_ref[...] + p1_ref[...]
    hn = jnp.maximum(
        jnp.dot(x_ref[...], At_ref[...], preferred_element_type=jnp.float32)
        + jnp.dot(s, Bt_ref[...], preferred_element_type=jnp.float32)
        + be_ref[...],
        0.0,
    )
    oh = (bt_ref[...] == lax.broadcasted_iota(jnp.int32, (BN, N_GRAPHS), 1)
          ).astype(jnp.float32)
    part = lax.dot_general(oh, hn, (((0,), (0,)), ((), ())),
                           preferred_element_type=jnp.float32)

    @pl.when(pl.program_id(0) == 0)
    def _():
        out_ref[...] = jnp.zeros_like(out_ref)

    out_ref[...] += part


def _tc_final(x, p0, p1, bt, At, Bt, be):
    return pl.pallas_call(
        _tc_final_body,
        grid=(GRID_N,),
        in_specs=[
            pl.BlockSpec((BN, D_NODE), lambda i: (i, 0)),
            pl.BlockSpec((BN, HIDDEN), lambda i: (i, 0)),
            pl.BlockSpec((BN, HIDDEN), lambda i: (i, 0)),
            pl.BlockSpec((BN, 1), lambda i: (i, 0)),
            pl.BlockSpec((D_NODE, HIDDEN), lambda i: (0, 0)),
            pl.BlockSpec((HIDDEN, HIDDEN), lambda i: (0, 0)),
            pl.BlockSpec((1, HIDDEN), lambda i: (0, 0)),
        ],
        out_specs=pl.BlockSpec((N_GRAPHS, HIDDEN), lambda i: (0, 0)),
        out_shape=jax.ShapeDtypeStruct((N_GRAPHS, HIDDEN), jnp.float32),
    )(x, p0, p1, bt, At, Bt, be)


def kernel(x, edge_index, edge_attr, batch, W_init, b_init, W1, b1, W2, b2,
           W3, b3, W_e2n, b_e2n):
    row = edge_index[0].astype(jnp.int32)
    col = edge_index[1].astype(jnp.int32)
    pad = ((0, IDXPAD - IDXROWS), (0, 0))
    row2d = jnp.pad(row.reshape(IDXROWS, K), pad)
    col2d = jnp.pad(col.reshape(IDXROWS, K), pad)

    Wxt = W_init[:, :D_NODE].T
    Wet = W_init[:, D_NODE:].T
    Wts = (W1.T, W2.T, W3.T)
    bis = (b1[None, :], b2[None, :], b3[None, :])
    zrows = jnp.zeros((ZR, HIDDEN), jnp.float32)

    xg = _sc_gather(x, row2d)
    h0, y = _tc_init(xg, edge_attr, Wxt, Wet, b_init[None, :], Wts[0])

    h3 = None
    for i in range(3):
        part = _sc_scatter(y, col2d, zrows)
        agg = _tc_add(part[0], part[1])
        g = _sc_gather(agg, row2d)
        if i < 2:
            y = _tc_conv(g, y, h0, bis[i], Wts[i + 1])
        else:
            h3 = _tc_convlast(g, y, h0, bis[i])

    part = _sc_scatter(h3, col2d, zrows)
    bt = batch.astype(jnp.int32).reshape(N_NODES, 1)
    emb = _tc_final(x, part[0], part[1], bt,
                    W_e2n[:, :D_NODE].T, W_e2n[:, D_NODE:].T, b_e2n[None, :])
    return emb


# speedup vs baseline: 3.4552x; 1.0427x over previous
import functools

import jax
import jax.numpy as jnp
from jax import lax
from jax.experimental import pallas as pl
from jax.experimental.pallas import tpu as pltpu
from jax.experimental.pallas import tpu_sc as plsc

N_NODES = 10000
N_EDGES = 320000
D_NODE = 128
D_EDGE = 16
HIDDEN = 128
N_GRAPHS = 64

NC = 2
NS = 16
NW = NC * NS
K = 128
IDXROWS = N_EDGES // K
IDXPAD = 2560
ROWS_PT = IDXPAD // NW
NPAIR = ROWS_PT // 2
RPT = 624
RTAIL = N_NODES - NS * RPT
ZR = 104
NZB = RPT // ZR

BE = 2000
GRID_E = N_EDGES // BE
BN = 2000
GRID_N = N_NODES // BN


def _mesh():
    return plsc.VectorSubcoreMesh(core_axis_name="c", subcore_axis_name="s")


def _worker_bounds(c, s):
    w = s * NC + c
    irow = w * ROWS_PT
    npair = jnp.where(w == NW - 1, (IDXROWS - (NW - 1) * ROWS_PT) // 2, NPAIR)
    return w, irow, npair


def _sc_gather_body(table, idx2d, out, idxb, bufa, bufb, gsa, gsb, ssa, ssb):
    c = lax.axis_index("c")
    s = lax.axis_index("s")
    w, irow, npair = _worker_bounds(c, s)
    pltpu.sync_copy(idx2d.at[pl.ds(pl.multiple_of(irow, 8), ROWS_PT)], idxb)

    def ebase(j):
        return pl.multiple_of((irow + j) * K, K)

    def start_gather(j, buf, sem):
        pltpu.async_copy(table.at[idxb.at[j]], buf, sem)

    def start_store(j, buf, sem):
        pltpu.async_copy(buf, out.at[pl.ds(ebase(j), K)], sem)

    def drain_gather(buf, sem):
        pltpu.make_async_copy(table.at[pl.ds(0, K)], buf, sem).wait()

    def drain_store(buf, sem):
        pltpu.make_async_copy(buf, out.at[pl.ds(0, K)], sem).wait()

    start_gather(0, bufa, gsa)

    def pair(j2, carry):
        p = 2 * j2
        q = p + 1

        @pl.when(j2 > 0)
        def _():
            drain_store(bufb, ssb)

        start_gather(q, bufb, gsb)
        drain_gather(bufa, gsa)
        start_store(p, bufa, ssa)

        @pl.when(j2 < npair - 1)
        def _():
            drain_store(bufa, ssa)
            start_gather(p + 2, bufa, gsa)

        drain_gather(bufb, gsb)
        start_store(q, bufb, ssb)
        return carry

    lax.fori_loop(0, npair, pair, 0)
    drain_store(bufa, ssa)
    drain_store(bufb, ssb)


def _sc_gather(table, idx2d):
    f = pl.kernel(
        _sc_gather_body,
        out_type=jax.ShapeDtypeStruct((N_EDGES, HIDDEN), jnp.float32),
        mesh=_mesh(),
        scratch_types=[
            pltpu.VMEM((ROWS_PT, K), jnp.int32),
            pltpu.VMEM((K, HIDDEN), jnp.float32),
            pltpu.VMEM((K, HIDDEN), jnp.float32),
            pltpu.SemaphoreType.DMA,
            pltpu.SemaphoreType.DMA,
            pltpu.SemaphoreType.DMA,
            pltpu.SemaphoreType.DMA,
        ],
    )
    return f(table, idx2d)


def _sc_scatter_body(vals, idx2d, zrows, out, idxb, bufa, bufb, vsa, vsb, acc):
    c = lax.axis_index("c")
    s = lax.axis_index("s")
    w, irow, npair = _worker_bounds(c, s)
    rbase = s * RPT

    pltpu.sync_copy(idx2d.at[pl.ds(pl.multiple_of(irow, 8), ROWS_PT)], idxb)
    pltpu.sync_copy(zrows, bufa.at[pl.ds(0, ZR)])
    for j in range(NZB):
        pltpu.sync_copy(bufa.at[pl.ds(0, ZR)], acc.at[pl.ds(rbase + j * ZR, ZR)])

    @pl.when(s == NS - 1)
    def _zero_tail():
        pltpu.sync_copy(bufa.at[pl.ds(0, RTAIL)], acc.at[pl.ds(NS * RPT, RTAIL)])

    plsc.subcore_barrier()

    def ebase(j):
        return pl.multiple_of((irow + j) * K, K)

    def start_load(j, buf, sem):
        pltpu.async_copy(vals.at[pl.ds(ebase(j), K)], buf, sem)

    def drain_load(buf, sem):
        pltpu.make_async_copy(vals.at[pl.ds(0, K)], buf, sem).wait()

    start_load(0, bufa, vsa)

    def pair(j2, carry):
        p = 2 * j2
        q = p + 1
        start_load(q, bufb, vsb)
        drain_load(bufa, vsa)
        pltpu.sync_copy(bufa, acc.at[idxb.at[p]], add=True)

        @pl.when(j2 < npair - 1)
        def _():
            start_load(p + 2, bufa, vsa)

        drain_load(bufb, vsb)
        pltpu.sync_copy(bufb, acc.at[idxb.at[q]], add=True)
        return carry

    lax.fori_loop(0, npair, pair, 0)
    plsc.subcore_barrier()

    for j in range(NZB):
        pltpu.sync_copy(acc.at[pl.ds(rbase + j * ZR, ZR)], bufa.at[pl.ds(0, ZR)])
        pltpu.sync_copy(bufa.at[pl.ds(0, ZR)], out.at[c, pl.ds(rbase + j * ZR, ZR)])

    @pl.when(s == NS - 1)
    def _write_tail():
        pltpu.sync_copy(acc.at[pl.ds(NS * RPT, RTAIL)], bufa.at[pl.ds(0, RTAIL)])
        pltpu.sync_copy(bufa.at[pl.ds(0, RTAIL)], out.at[c, pl.ds(NS * RPT, RTAIL)])


def _sc_scatter(vals, idx2d, zrows):
    f = pl.kernel(
        _sc_scatter_body,
        out_type=jax.ShapeDtypeStruct((NC, N_NODES, HIDDEN), jnp.float32),
        mesh=_mesh(),
        scratch_types=[
            pltpu.VMEM((ROWS_PT, K), jnp.int32),
            pltpu.VMEM((K, HIDDEN), jnp.float32),
            pltpu.VMEM((K, HIDDEN), jnp.float32),
            pltpu.SemaphoreType.DMA,
            pltpu.SemaphoreType.DMA,
            pltpu.VMEM_SHARED((N_NODES, HIDDEN), jnp.float32),
        ],
    )
    return f(vals, idx2d, zrows)


def _pair_swap(y):
    even = (lax.broadcasted_iota(jnp.int32, y.shape, 0) & 1) == 0
    return jnp.where(even, pltpu.roll(y, y.shape[0] - 1, 0),
                     pltpu.roll(y, 1, 0))


def _tc_init_body(xg_ref, ea_ref, Wx_ref, We_ref, b_ref, W1_ref, h0_ref, y_ref):
    h0 = jnp.maximum(
        jnp.dot(xg_ref[...], Wx_ref[...], preferred_element_type=jnp.float32)
        + jnp.dot(ea_ref[...].astype(jnp.float32), We_ref[...],
                  preferred_element_type=jnp.float32)
        + b_ref[...],
        0.0,
    )
    h0_ref[...] = h0.astype(jnp.bfloat16)
    y_ref[...] = jnp.dot(h0, W1_ref[...], preferred_element_type=jnp.float32)


def _tc_init(xg, ea, Wxt, Wet, bi, W1t):
    return pl.pallas_call(
        _tc_init_body,
        grid=(GRID_E,),
        in_specs=[
            pl.BlockSpec((BE, D_NODE), lambda i: (i, 0)),
            pl.BlockSpec((BE, D_EDGE), lambda i: (i, 0)),
            pl.BlockSpec((D_NODE, HIDDEN), lambda i: (0, 0)),
            pl.BlockSpec((D_EDGE, HIDDEN), lambda i: (0, 0)),
            pl.BlockSpec((1, HIDDEN), lambda i: (0, 0)),
            pl.BlockSpec((HIDDEN, HIDDEN), lambda i: (0, 0)),
        ],
        out_specs=[
            pl.BlockSpec((BE, HIDDEN), lambda i: (i, 0)),
            pl.BlockSpec((BE, HIDDEN), lambda i: (i, 0)),
        ],
        out_shape=[
            jax.ShapeDtypeStruct((N_EDGES, HIDDEN), jnp.bfloat16),
            jax.ShapeDtypeStruct((N_EDGES, HIDDEN), jnp.float32),
        ],
    )(xg, ea, Wxt, Wet, bi, W1t)


def _tc_conv_body(g_ref, y_ref, h0_ref, b_ref, W_ref, out_ref):
    h = jnp.maximum(
        g_ref[...] - _pair_swap(y_ref[...]) + b_ref[...]
        + h0_ref[...].astype(jnp.float32), 0.0)
    out_ref[...] = jnp.dot(h, W_ref[...], preferred_element_type=jnp.float32)


def _tc_conv(g, y, h0, b, Wt):
    return pl.pallas_call(
        _tc_conv_body,
        grid=(GRID_E,),
        in_specs=[
            pl.BlockSpec((BE, HIDDEN), lambda i: (i, 0)),
            pl.BlockSpec((BE, HIDDEN), lambda i: (i, 0)),
            pl.BlockSpec((BE, HIDDEN), lambda i: (i, 0)),
            pl.BlockSpec((1, HIDDEN), lambda i: (0, 0)),
            pl.BlockSpec((HIDDEN, HIDDEN), lambda i: (0, 0)),
        ],
        out_specs=pl.BlockSpec((BE, HIDDEN), lambda i: (i, 0)),
        out_shape=jax.ShapeDtypeStruct((N_EDGES, HIDDEN), jnp.float32),
    )(g, y, h0, b, Wt)


def _tc_convlast_body(g_ref, y_ref, h0_ref, b_ref, out_ref):
    out_ref[...] = jnp.maximum(
        g_ref[...] - _pair_swap(y_ref[...]) + b_ref[...]
        + h0_ref[...].astype(jnp.float32), 0.0)


def _tc_convlast(g, y, h0, b):
    return pl.pallas_call(
        _tc_convlast_body,
        grid=(GRID_E,),
        in_specs=[
            pl.BlockSpec((BE, HIDDEN), lambda i: (i, 0)),
            pl.BlockSpec((BE, HIDDEN), lambda i: (i, 0)),
            pl.BlockSpec((BE, HIDDEN), lambda i: (i, 0)),
            pl.BlockSpec((1, HIDDEN), lambda i: (0, 0)),
        ],
        out_specs=pl.BlockSpec((BE, HIDDEN), lambda i: (i, 0)),
        out_shape=jax.ShapeDtypeStruct((N_EDGES, HIDDEN), jnp.float32),
    )(g, y, h0, b)


def _tc_add_body(a_ref, b_ref, o_ref):
    o_ref[...] = a_ref[...] + b_ref[...]


def _tc_add(a, b):
    return pl.pallas_call(
        _tc_add_body,
        grid=(GRID_N,),
        in_specs=[
            pl.BlockSpec((BN, HIDDEN), lambda i: (i, 0)),
            pl.BlockSpec((BN, HIDDEN), lambda i: (i, 0)),
        ],
        out_specs=pl.BlockSpec((BN, HIDDEN), lambda i: (i, 0)),
        out_shape=jax.ShapeDtypeStruct((N_NODES, HIDDEN), jnp.float32),
    )(a, b)


def _tc_final_body(x_ref, p0_ref, p1_ref, bt_ref, At_ref, Bt_ref, be_ref, out_ref):
    s = p0_ref[...] + p1_ref[...]
    hn = jnp.maximum(
        jnp.dot(x_ref[...], At_ref[...], preferred_element_type=jnp.float32)
        + jnp.dot(s, Bt_ref[...], preferred_element_type=jnp.float32)
        + be_ref[...],
        0.0,
    )
    oh = (bt_ref[...] == lax.broadcasted_iota(jnp.int32, (BN, N_GRAPHS), 1)
          ).astype(jnp.float32)
    part = lax.dot_general(oh, hn, (((0,), (0,)), ((), ())),
                           preferred_element_type=jnp.float32)

    @pl.when(pl.program_id(0) == 0)
    def _():
        out_ref[...] = jnp.zeros_like(out_ref)

    out_ref[...] += part


def _tc_final(x, p0, p1, bt, At, Bt, be):
    return pl.pallas_call(
        _tc_final_body,
        grid=(GRID_N,),
        in_specs=[
            pl.BlockSpec((BN, D_NODE), lambda i: (i, 0)),
            pl.BlockSpec((BN, HIDDEN), lambda i: (i, 0)),
            pl.BlockSpec((BN, HIDDEN), lambda i: (i, 0)),
            pl.BlockSpec((BN, 1), lambda i: (i, 0)),
            pl.BlockSpec((D_NODE, HIDDEN), lambda i: (0, 0)),
            pl.BlockSpec((HIDDEN, HIDDEN), lambda i: (0, 0)),
            pl.BlockSpec((1, HIDDEN), lambda i: (0, 0)),
        ],
        out_specs=pl.BlockSpec((N_GRAPHS, HIDDEN), lambda i: (0, 0)),
        out_shape=jax.ShapeDtypeStruct((N_GRAPHS, HIDDEN), jnp.float32),
    )(x, p0, p1, bt, At, Bt, be)


def kernel(x, edge_index, edge_attr, batch, W_init, b_init, W1, b1, W2, b2,
           W3, b3, W_e2n, b_e2n):
    row = edge_index[0].astype(jnp.int32)
    col = edge_index[1].astype(jnp.int32)
    pad = ((0, IDXPAD - IDXROWS), (0, 0))
    row2d = jnp.pad(row.reshape(IDXROWS, K), pad)
    col2d = jnp.pad(col.reshape(IDXROWS, K), pad)

    Wxt = W_init[:, :D_NODE].T
    Wet = W_init[:, D_NODE:].T
    Wts = (W1.T, W2.T, W3.T)
    bis = (b1[None, :], b2[None, :], b3[None, :])
    zrows = jnp.zeros((ZR, HIDDEN), jnp.float32)

    xg = _sc_gather(x, row2d)
    h0, y = _tc_init(xg, edge_attr.astype(jnp.bfloat16), Wxt, Wet,
                     b_init[None, :], Wts[0])

    h3 = None
    for i in range(3):
        part = _sc_scatter(y, col2d, zrows)
        agg = _tc_add(part[0], part[1])
        g = _sc_gather(agg, row2d)
        if i < 2:
            y = _tc_conv(g, y, h0, bis[i], Wts[i + 1])
        else:
            h3 = _tc_convlast(g, y, h0, bis[i])

    part = _sc_scatter(h3, col2d, zrows)
    bt = batch.astype(jnp.int32).reshape(N_NODES, 1)
    emb = _tc_final(x, part[0], part[1], bt,
                    W_e2n[:, :D_NODE].T, W_e2n[:, D_NODE:].T, b_e2n[None, :])
    return emb
